# baseline (speedup 1.0000x reference)
import jax
import jax.numpy as jnp
from jax import lax
from jax.experimental import pallas as pl
from jax.experimental.pallas import tpu as pltpu
from jax.experimental.pallas import tpu_sc as plsc

_NUM_CLASSES = 1000000
_V = _NUM_CLASSES + 1
_D = 64
_B = 16384
_DROPOUT_PROB = 0.1

_NWORK = 32
_LANES = 16
_WIN = 256
_TAIL_BASE = 999936
_TAIL_N = _V - _TAIL_BASE
_NWIN = _TAIL_BASE // _WIN
_CAP = 704
_PCAP = 256
_CHUNK = 2048
_NCHUNK = _B // _CHUNK
_Q, _R = divmod(_NWIN, _NWORK)
_FBIT = 1 << 30
_LMASK = _FBIT - 1
_CFG_OFF = (_NUM_CLASSES - _TAIL_BASE) * _D


def _body(labels_hbm, tableT_hbm, tail_hbm, out_hbm,
          labc0_v, labc1_v, mlab_v, mpos_v, tail_v,
          win0_v, win1_v, win2_v, win3_v, rows_v,
          pcol_v, pslot_v, ppos_v,
          sem0, sem1, sem2, sem3, semo, semi):
    wid = lax.axis_index("s") * 2 + lax.axis_index("c")
    iota = lax.iota(jnp.int32, _LANES)

    nw = _Q + jnp.where(wid < _R, 1, 0).astype(jnp.int32)
    w0 = wid * _Q + jnp.minimum(wid, _R)
    lo = w0 * _WIN
    hi = jnp.where(wid == _NWORK - 1, jnp.int32(2**29), (w0 + nw) * _WIN)

    bufs = (win0_v, win1_v, win2_v, win3_v)
    sems = (sem0, sem1, sem2, sem3)

    def fire(widx, win, sem):
        wb_lanes = pl.multiple_of((w0 + widx) * _WIN, _WIN)
        for g in range(_D // 8):
            pltpu.async_copy(
                tableT_hbm.at[pl.ds(g * 8, 8), pl.ds(wb_lanes, _WIN)],
                win.at[pl.ds(g * 8, 8)],
                sem,
            )

    def wait_win(widx, win, sem):
        wb_lanes = pl.multiple_of((w0 + widx) * _WIN, _WIN)
        pltpu.make_async_copy(
            tableT_hbm.at[:, pl.ds(wb_lanes, _WIN)], win, sem
        ).wait()

    for j in range(4):
        @pl.when(nw > j)
        def _(j=j):
            fire(j, bufs[j], sems[j])

    cp_tail = pltpu.async_copy(tail_hbm, tail_v, semi)
    pltpu.async_copy(labels_hbm.at[pl.ds(0, _CHUNK)], labc0_v, semi)
    cp_tail.wait()

    def scan_group(base, src_v):
        def g_body(g, cnt):
            raw = src_v[pl.ds(g * _LANES, _LANES)]
            lab = raw & _LMASK
            m = jnp.logical_and(lab >= lo, lab < hi)
            plsc.store_compressed(mlab_v.at[pl.ds(cnt, _LANES)], raw, mask=m)
            pos = iota + base + g * _LANES
            plsc.store_compressed(mpos_v.at[pl.ds(cnt, _LANES)], pos, mask=m)
            return cnt + plsc.all_reduce_population_count(m)[0]
        return g_body

    cnt = jnp.int32(0)
    for c in range(_NCHUNK):
        cur = labc0_v if c % 2 == 0 else labc1_v
        nxt = labc1_v if c % 2 == 0 else labc0_v
        pltpu.make_async_copy(
            labels_hbm.at[pl.ds(c * _CHUNK, _CHUNK)], cur, semi
        ).wait()
        if c + 1 < _NCHUNK:
            pltpu.async_copy(
                labels_hbm.at[pl.ds((c + 1) * _CHUNK, _CHUNK)], nxt, semi
            )
        cnt = lax.fori_loop(
            0, _CHUNK // _LANES, scan_group(c * _CHUNK, cur), cnt,
            unroll=4,
        )

    jmax = (cnt + _LANES - 1) // _LANES

    def emit_row(slot, pos):
        pltpu.async_copy(
            rows_v.at[pl.ds(slot * _D, _D)],
            out_hbm.at[pl.ds(pos * _D, _D)],
            semo,
        )

    def process_window(widx, win):
        wb = (w0 + widx) * _WIN

        def jbody(j, pcnt):
            valid = (iota + j * _LANES) < cnt
            raw = mlab_v[pl.ds(j * _LANES, _LANES)]
            ml = raw & _LMASK
            inw = valid & (raw < _FBIT) & (ml >= wb) & (ml < wb + _WIN)
            plsc.store_compressed(
                pcol_v.at[pl.ds(pcnt, _LANES)], ml - wb, mask=inw
            )
            plsc.store_compressed(
                pslot_v.at[pl.ds(pcnt, _LANES)], iota + j * _LANES, mask=inw
            )
            plsc.store_compressed(
                ppos_v.at[pl.ds(pcnt, _LANES)],
                mpos_v[pl.ds(j * _LANES, _LANES)],
                mask=inw,
            )
            return pcnt + plsc.all_reduce_population_count(inw)[0]

        pcnt = lax.fori_loop(0, _CAP // _LANES, jbody, jnp.int32(0), unroll=4)

        def ebody(e, carry):
            em = (iota + e * _LANES) < pcnt
            cols = pcol_v[pl.ds(e * _LANES, _LANES)]
            slots = pslot_v[pl.ds(e * _LANES, _LANES)]
            poss = ppos_v[pl.ds(e * _LANES, _LANES)]
            soff = slots * _D
            for c in range(_D):
                rowv = jnp.zeros((_LANES,), jnp.int32) + c
                vals = plsc.load_gather(win, [rowv, cols], mask=em)
                plsc.store_scatter(rows_v, [soff + c], vals, mask=em)
            em32 = em.astype(jnp.int32)
            for k in range(_LANES):
                @pl.when(em32[k] > 0)
                def _(k=k):
                    emit_row(slots[k], poss[k])
            return carry

        lax.fori_loop(0, (pcnt + _LANES - 1) // _LANES, ebody, 0)

    def quad(ii, carry):
        base = ii * 4
        for b in range(4):
            widx = base + b

            @pl.when(widx < nw)
            def _(widx=widx, b=b):
                wait_win(widx, bufs[b], sems[b])
                process_window(widx, bufs[b])

                @pl.when(widx + 4 < nw)
                def _():
                    fire(widx + 4, bufs[b], sems[b])

        return carry

    lax.fori_loop(0, (nw + 3) // 4, quad, 0)

    def tbody(j, carry):
        valid = (iota + j * _LANES) < cnt
        raw = mlab_v[pl.ds(j * _LANES, _LANES)]
        ml = raw & _LMASK
        int_ = valid & ((ml >= _TAIL_BASE) | (raw >= _FBIT))
        pc = plsc.all_reduce_population_count(int_)[0]

        @pl.when(pc > 0)
        def _():
            int32_ = int_.astype(jnp.int32)
            flv = (raw >> 30).astype(jnp.int32)
            for k in range(_LANES):
                @pl.when(int32_[k] > 0)
                def _(k=k):
                    slot = j * _LANES + k
                    src = jnp.where(flv[k] != 0, _NUM_CLASSES, ml[k]) - _TAIL_BASE
                    for g in range(_D // _LANES):
                        rows_v[pl.ds(slot * _D + g * _LANES, _LANES)] = tail_v[
                            pl.ds(src * _D + g * _LANES, _LANES)
                        ]
                    emit_row(slot, mpos_v[pl.ds(j * _LANES, _LANES)][k])

        return carry

    lax.fori_loop(0, jmax, tbody, 0)

    def dbody(i, carry):
        pltpu.make_async_copy(
            tableT_hbm.at[0, pl.ds(0, _D)], rows_v.at[pl.ds(0, _D)], semo
        ).wait()
        return carry

    lax.fori_loop(0, cnt, dbody, 0)


def kernel(labels, train, embedding_table):
    b = labels.shape[0]
    rand_drop = jax.random.uniform(jax.random.key(1), (b,)) < _DROPOUT_PROB
    use_dropout = jnp.logical_and(_DROPOUT_PROB > 0, train != 0)
    flags = jnp.logical_and(rand_drop, use_dropout).astype(jnp.int32)
    packed = (labels.astype(jnp.int32) | (flags << 30)).astype(jnp.int32)
    tableT = embedding_table.T
    tail = embedding_table[_TAIL_BASE:].reshape(-1)

    mesh = plsc.VectorSubcoreMesh(core_axis_name="c", subcore_axis_name="s")
    out = pl.kernel(
        _body,
        out_type=jax.ShapeDtypeStruct((_B * _D,), jnp.float32),
        mesh=mesh,
        compiler_params=pltpu.CompilerParams(needs_layout_passes=False),
        scratch_types=[
            pltpu.VMEM((_CHUNK,), jnp.int32),
            pltpu.VMEM((_CHUNK,), jnp.int32),
            pltpu.VMEM((_CAP + _LANES,), jnp.int32),
            pltpu.VMEM((_CAP + _LANES,), jnp.int32),
            pltpu.VMEM((_TAIL_N * _D,), jnp.float32),
            pltpu.VMEM((_D, _WIN), jnp.float32),
            pltpu.VMEM((_D, _WIN), jnp.float32),
            pltpu.VMEM((_D, _WIN), jnp.float32),
            pltpu.VMEM((_D, _WIN), jnp.float32),
            pltpu.VMEM((_CAP * _D,), jnp.float32),
            pltpu.VMEM((_PCAP + _LANES,), jnp.int32),
            pltpu.VMEM((_PCAP + _LANES,), jnp.int32),
            pltpu.VMEM((_PCAP + _LANES,), jnp.int32),
            pltpu.SemaphoreType.DMA,
            pltpu.SemaphoreType.DMA,
            pltpu.SemaphoreType.DMA,
            pltpu.SemaphoreType.DMA,
            pltpu.SemaphoreType.DMA,
            pltpu.SemaphoreType.DMA,
        ],
    )(packed, tableT, tail)
    return out.reshape(b, _D)


# speedup vs baseline: 1.1151x; 1.1151x over previous
import jax
import jax.numpy as jnp
from jax import lax
from jax.experimental import pallas as pl
from jax.experimental.pallas import tpu as pltpu
from jax.experimental.pallas import tpu_sc as plsc

_NUM_CLASSES = 1000000
_V = _NUM_CLASSES + 1
_D = 64
_B = 16384
_DROPOUT_PROB = 0.1

_NWORK = 32
_LANES = 16
_WIN = 256
_TAIL_BASE = 999936
_TAIL_N = _V - _TAIL_BASE
_NWIN = _TAIL_BASE // _WIN
_CAP = 704
_PCAP = 256
_CHUNK = 2048
_NCHUNK = _B // _CHUNK
_Q, _R = divmod(_NWIN, _NWORK)
_FBIT = 1 << 30
_LMASK = _FBIT - 1
_CFG_OFF = (_NUM_CLASSES - _TAIL_BASE) * _D


def _body(labels_hbm, tableT_hbm, tail_hbm, out_hbm,
          labc0_v, labc1_v, mlab_v, mpos_v, tail_v,
          win0_v, win1_v, win2_v, win3_v, rows_v,
          pcol_v, pslot_v, ppos_v,
          sem0, sem1, sem2, sem3, semo, semi):
    wid = lax.axis_index("s") * 2 + lax.axis_index("c")
    iota = lax.iota(jnp.int32, _LANES)

    nw = _Q + jnp.where(wid < _R, 1, 0).astype(jnp.int32)
    w0 = wid * _Q + jnp.minimum(wid, _R)
    lo = w0 * _WIN
    hi = jnp.where(wid == _NWORK - 1, jnp.int32(2**29), (w0 + nw) * _WIN)

    bufs = (win0_v, win1_v, win2_v, win3_v)
    sems = (sem0, sem1, sem2, sem3)

    def fire(widx, win, sem):
        wb_lanes = pl.multiple_of((w0 + widx) * _WIN, _WIN)
        for g in range(_D // 8):
            pltpu.async_copy(
                tableT_hbm.at[pl.ds(g * 8, 8), pl.ds(wb_lanes, _WIN)],
                win.at[pl.ds(g * 8, 8)],
                sem,
            )

    def wait_win(widx, win, sem):
        wb_lanes = pl.multiple_of((w0 + widx) * _WIN, _WIN)
        pltpu.make_async_copy(
            tableT_hbm.at[:, pl.ds(wb_lanes, _WIN)], win, sem
        ).wait()

    for j in range(4):
        @pl.when(nw > j)
        def _(j=j):
            fire(j, bufs[j], sems[j])

    cp_tail = pltpu.async_copy(tail_hbm, tail_v, semi)
    pltpu.async_copy(labels_hbm.at[pl.ds(0, _CHUNK)], labc0_v, semi)
    cp_tail.wait()

    def scan_group(base, src_v):
        def g_body(g, cnt):
            raw = src_v[pl.ds(g * _LANES, _LANES)]
            lab = raw & _LMASK
            m = jnp.logical_and(lab >= lo, lab < hi)
            plsc.store_compressed(mlab_v.at[pl.ds(cnt, _LANES)], raw, mask=m)
            pos = iota + base + g * _LANES
            plsc.store_compressed(mpos_v.at[pl.ds(cnt, _LANES)], pos, mask=m)
            return cnt + plsc.all_reduce_population_count(m)[0]
        return g_body

    cnt = jnp.int32(0)
    for c in range(_NCHUNK):
        cur = labc0_v if c % 2 == 0 else labc1_v
        nxt = labc1_v if c % 2 == 0 else labc0_v
        pltpu.make_async_copy(
            labels_hbm.at[pl.ds(c * _CHUNK, _CHUNK)], cur, semi
        ).wait()
        if c + 1 < _NCHUNK:
            pltpu.async_copy(
                labels_hbm.at[pl.ds((c + 1) * _CHUNK, _CHUNK)], nxt, semi
            )
        cnt = lax.fori_loop(
            0, _CHUNK // _LANES, scan_group(c * _CHUNK, cur), cnt,
            unroll=4,
        )

    jmax = (cnt + _LANES - 1) // _LANES

    def emit_row(slot, pos):
        pltpu.async_copy(
            rows_v.at[pl.ds(slot * _D, _D)],
            out_hbm.at[pl.ds(pos * _D, _D)],
            semo,
        )

    def process_window(widx, win):
        wb = (w0 + widx) * _WIN

        def jbody(j, pcnt):
            valid = (iota + j * _LANES) < cnt
            raw = mlab_v[pl.ds(j * _LANES, _LANES)]
            ml = raw & _LMASK
            inw = valid & (raw < _FBIT) & (ml >= wb) & (ml < wb + _WIN)
            plsc.store_compressed(
                pcol_v.at[pl.ds(pcnt, _LANES)], ml - wb, mask=inw
            )
            plsc.store_compressed(
                pslot_v.at[pl.ds(pcnt, _LANES)], iota + j * _LANES, mask=inw
            )
            plsc.store_compressed(
                ppos_v.at[pl.ds(pcnt, _LANES)],
                mpos_v[pl.ds(j * _LANES, _LANES)],
                mask=inw,
            )
            return pcnt + plsc.all_reduce_population_count(inw)[0]

        pcnt = lax.fori_loop(0, jmax, jbody, jnp.int32(0))

        def ebody(e, carry):
            em = (iota + e * _LANES) < pcnt
            cols = pcol_v[pl.ds(e * _LANES, _LANES)]
            slots = pslot_v[pl.ds(e * _LANES, _LANES)]
            poss = ppos_v[pl.ds(e * _LANES, _LANES)]
            soff = slots * _D
            for c in range(_D):
                rowv = jnp.zeros((_LANES,), jnp.int32) + c
                vals = plsc.load_gather(win, [rowv, cols], mask=em)
                plsc.store_scatter(rows_v, [soff + c], vals, mask=em)
            em32 = em.astype(jnp.int32)
            for k in range(_LANES):
                @pl.when(em32[k] > 0)
                def _(k=k):
                    emit_row(slots[k], poss[k])
            return carry

        lax.fori_loop(0, (pcnt + _LANES - 1) // _LANES, ebody, 0)

    def quad(ii, carry):
        base = ii * 4
        for b in range(4):
            widx = base + b

            @pl.when(widx < nw)
            def _(widx=widx, b=b):
                wait_win(widx, bufs[b], sems[b])
                process_window(widx, bufs[b])

                @pl.when(widx + 4 < nw)
                def _():
                    fire(widx + 4, bufs[b], sems[b])

        return carry

    lax.fori_loop(0, (nw + 3) // 4, quad, 0)

    def tbody(j, carry):
        valid = (iota + j * _LANES) < cnt
        raw = mlab_v[pl.ds(j * _LANES, _LANES)]
        ml = raw & _LMASK
        int_ = valid & ((ml >= _TAIL_BASE) | (raw >= _FBIT))
        pc = plsc.all_reduce_population_count(int_)[0]

        @pl.when(pc > 0)
        def _():
            int32_ = int_.astype(jnp.int32)
            flv = (raw >> 30).astype(jnp.int32)
            for k in range(_LANES):
                @pl.when(int32_[k] > 0)
                def _(k=k):
                    slot = j * _LANES + k
                    src = jnp.where(flv[k] != 0, _NUM_CLASSES, ml[k]) - _TAIL_BASE
                    for g in range(_D // _LANES):
                        rows_v[pl.ds(slot * _D + g * _LANES, _LANES)] = tail_v[
                            pl.ds(src * _D + g * _LANES, _LANES)
                        ]
                    emit_row(slot, mpos_v[pl.ds(j * _LANES, _LANES)][k])

        return carry

    lax.fori_loop(0, jmax, tbody, 0)

    def dbody(i, carry):
        pltpu.make_async_copy(
            tableT_hbm.at[0, pl.ds(0, _D)], rows_v.at[pl.ds(0, _D)], semo
        ).wait()
        return carry

    lax.fori_loop(0, cnt, dbody, 0)


def kernel(labels, train, embedding_table):
    b = labels.shape[0]
    rand_drop = jax.random.uniform(jax.random.key(1), (b,)) < _DROPOUT_PROB
    use_dropout = jnp.logical_and(_DROPOUT_PROB > 0, train != 0)
    flags = jnp.logical_and(rand_drop, use_dropout).astype(jnp.int32)
    packed = (labels.astype(jnp.int32) | (flags << 30)).astype(jnp.int32)
    tableT = embedding_table.T
    tail = embedding_table[_TAIL_BASE:].reshape(-1)

    mesh = plsc.VectorSubcoreMesh(core_axis_name="c", subcore_axis_name="s")
    out = pl.kernel(
        _body,
        out_type=jax.ShapeDtypeStruct((_B * _D,), jnp.float32),
        mesh=mesh,
        compiler_params=pltpu.CompilerParams(needs_layout_passes=False),
        scratch_types=[
            pltpu.VMEM((_CHUNK,), jnp.int32),
            pltpu.VMEM((_CHUNK,), jnp.int32),
            pltpu.VMEM((_CAP + _LANES,), jnp.int32),
            pltpu.VMEM((_CAP + _LANES,), jnp.int32),
            pltpu.VMEM((_TAIL_N * _D,), jnp.float32),
            pltpu.VMEM((_D, _WIN), jnp.float32),
            pltpu.VMEM((_D, _WIN), jnp.float32),
            pltpu.VMEM((_D, _WIN), jnp.float32),
            pltpu.VMEM((_D, _WIN), jnp.float32),
            pltpu.VMEM((_CAP * _D,), jnp.float32),
            pltpu.VMEM((_PCAP + _LANES,), jnp.int32),
            pltpu.VMEM((_PCAP + _LANES,), jnp.int32),
            pltpu.VMEM((_PCAP + _LANES,), jnp.int32),
            pltpu.SemaphoreType.DMA,
            pltpu.SemaphoreType.DMA,
            pltpu.SemaphoreType.DMA,
            pltpu.SemaphoreType.DMA,
            pltpu.SemaphoreType.DMA,
            pltpu.SemaphoreType.DMA,
        ],
    )(packed, tableT, tail)
    return out.reshape(b, _D)


# speedup vs baseline: 1.2139x; 1.0886x over previous
import jax
import jax.numpy as jnp
from jax import lax
from jax.experimental import pallas as pl
from jax.experimental.pallas import tpu as pltpu
from jax.experimental.pallas import tpu_sc as plsc

_NUM_CLASSES = 1000000
_V = _NUM_CLASSES + 1
_D = 64
_B = 16384
_DROPOUT_PROB = 0.1

_NWORK = 32
_LANES = 16
_WIN = 256
_TAIL_BASE = 999936
_TAIL_N = _V - _TAIL_BASE
_NWIN = _TAIL_BASE // _WIN
_CAP = 704
_PCAP = 256
_CHUNK = 2048
_NCHUNK = _B // _CHUNK
_Q, _R = divmod(_NWIN, _NWORK)
_FBIT = 1 << 30
_LMASK = _FBIT - 1
_CFG_OFF = (_NUM_CLASSES - _TAIL_BASE) * _D


def _body(labels_hbm, tableT_hbm, tail_hbm, out_hbm,
          labc0_v, labc1_v, mlab_v, mpos_v, tail_v,
          win0_v, win1_v, win2_v, win3_v, rows_v,
          pcol_v, pslot_v,
          sem0, sem1, sem2, sem3, semo, semi):
    wid = lax.axis_index("s") * 2 + lax.axis_index("c")
    iota = lax.iota(jnp.int32, _LANES)

    nw = _Q + jnp.where(wid < _R, 1, 0).astype(jnp.int32)
    w0 = wid * _Q + jnp.minimum(wid, _R)
    lo = w0 * _WIN
    hi = jnp.where(wid == _NWORK - 1, jnp.int32(2**29), (w0 + nw) * _WIN)

    bufs = (win0_v, win1_v, win2_v, win3_v)
    sems = (sem0, sem1, sem2, sem3)

    def fire(widx, win, sem):
        wb_lanes = pl.multiple_of((w0 + widx) * _WIN, _WIN)
        for g in range(_D // 8):
            pltpu.async_copy(
                tableT_hbm.at[pl.ds(g * 8, 8), pl.ds(wb_lanes, _WIN)],
                win.at[pl.ds(g * 8, 8)],
                sem,
            )

    def wait_win(widx, win, sem):
        wb_lanes = pl.multiple_of((w0 + widx) * _WIN, _WIN)
        pltpu.make_async_copy(
            tableT_hbm.at[:, pl.ds(wb_lanes, _WIN)], win, sem
        ).wait()

    for j in range(4):
        @pl.when(nw > j)
        def _(j=j):
            fire(j, bufs[j], sems[j])

    cp_tail = pltpu.async_copy(tail_hbm, tail_v, semi)
    pltpu.async_copy(labels_hbm.at[pl.ds(0, _CHUNK)], labc0_v, semi)
    cp_tail.wait()

    def scan_group(base, src_v):
        def g_body(g, cnt):
            raw = src_v[pl.ds(g * _LANES, _LANES)]
            lab = raw & _LMASK
            m = jnp.logical_and(lab >= lo, lab < hi)
            plsc.store_compressed(mlab_v.at[pl.ds(cnt, _LANES)], raw, mask=m)
            pos = iota + base + g * _LANES
            plsc.store_compressed(mpos_v.at[pl.ds(cnt, _LANES)], pos, mask=m)
            return cnt + plsc.all_reduce_population_count(m)[0]
        return g_body

    cnt = jnp.int32(0)
    for c in range(_NCHUNK):
        cur = labc0_v if c % 2 == 0 else labc1_v
        nxt = labc1_v if c % 2 == 0 else labc0_v
        pltpu.make_async_copy(
            labels_hbm.at[pl.ds(c * _CHUNK, _CHUNK)], cur, semi
        ).wait()
        if c + 1 < _NCHUNK:
            pltpu.async_copy(
                labels_hbm.at[pl.ds((c + 1) * _CHUNK, _CHUNK)], nxt, semi
            )
        cnt = lax.fori_loop(
            0, _CHUNK // _LANES, scan_group(c * _CHUNK, cur), cnt,
            unroll=4,
        )

    jmax = (cnt + _LANES - 1) // _LANES
    mlab_v[pl.ds(cnt, _LANES)] = jnp.zeros((_LANES,), jnp.int32) + (1 << 29)

    def emit_row(slot, pos):
        pltpu.async_copy(
            rows_v.at[pl.ds(slot * _D, _D)],
            out_hbm.at[pl.ds(pos * _D, _D)],
            semo,
        )

    def process_window(widx, win):
        wb = (w0 + widx) * _WIN

        def jbody(j, pcnt):
            raw = mlab_v[pl.ds(j * _LANES, _LANES)]
            ml = raw & _LMASK
            inw = (raw < _FBIT) & (ml >= wb) & (ml < wb + _WIN)
            plsc.store_compressed(
                pcol_v.at[pl.ds(pcnt, _LANES)], ml - wb, mask=inw
            )
            plsc.store_compressed(
                pslot_v.at[pl.ds(pcnt, _LANES)], iota + j * _LANES, mask=inw
            )
            return pcnt + plsc.all_reduce_population_count(inw)[0]

        pcnt = lax.fori_loop(0, jmax, jbody, jnp.int32(0))

        def ebody(e, carry):
            em = (iota + e * _LANES) < pcnt
            cols = pcol_v[pl.ds(e * _LANES, _LANES)]
            slots = pslot_v[pl.ds(e * _LANES, _LANES)]
            poss = plsc.load_gather(mpos_v, [slots], mask=em)
            soff = slots * _D
            for c in range(_D):
                rowv = jnp.zeros((_LANES,), jnp.int32) + c
                vals = plsc.load_gather(win, [rowv, cols], mask=em)
                plsc.store_scatter(rows_v, [soff + c], vals, mask=em)
            em32 = em.astype(jnp.int32)
            for k in range(_LANES):
                @pl.when(em32[k] > 0)
                def _(k=k):
                    emit_row(slots[k], poss[k])
            return carry

        lax.fori_loop(0, (pcnt + _LANES - 1) // _LANES, ebody, 0)

    def quad(ii, carry):
        base = ii * 4
        for b in range(4):
            widx = base + b

            @pl.when(widx < nw)
            def _(widx=widx, b=b):
                wait_win(widx, bufs[b], sems[b])
                process_window(widx, bufs[b])

                @pl.when(widx + 4 < nw)
                def _():
                    fire(widx + 4, bufs[b], sems[b])

        return carry

    lax.fori_loop(0, (nw + 3) // 4, quad, 0)

    def tbody(j, carry):
        valid = (iota + j * _LANES) < cnt
        raw = mlab_v[pl.ds(j * _LANES, _LANES)]
        ml = raw & _LMASK
        int_ = valid & ((ml >= _TAIL_BASE) | (raw >= _FBIT))
        pc = plsc.all_reduce_population_count(int_)[0]

        @pl.when(pc > 0)
        def _():
            int32_ = int_.astype(jnp.int32)
            flv = (raw >> 30).astype(jnp.int32)
            for k in range(_LANES):
                @pl.when(int32_[k] > 0)
                def _(k=k):
                    slot = j * _LANES + k
                    src = jnp.where(flv[k] != 0, _NUM_CLASSES, ml[k]) - _TAIL_BASE
                    for g in range(_D // _LANES):
                        rows_v[pl.ds(slot * _D + g * _LANES, _LANES)] = tail_v[
                            pl.ds(src * _D + g * _LANES, _LANES)
                        ]
                    emit_row(slot, mpos_v[pl.ds(j * _LANES, _LANES)][k])

        return carry

    lax.fori_loop(0, jmax, tbody, 0)

    def dbody(i, carry):
        pltpu.make_async_copy(
            tableT_hbm.at[0, pl.ds(0, _D)], rows_v.at[pl.ds(0, _D)], semo
        ).wait()
        return carry

    lax.fori_loop(0, cnt, dbody, 0)


def kernel(labels, train, embedding_table):
    b = labels.shape[0]
    rand_drop = jax.random.uniform(jax.random.key(1), (b,)) < _DROPOUT_PROB
    use_dropout = jnp.logical_and(_DROPOUT_PROB > 0, train != 0)
    flags = jnp.logical_and(rand_drop, use_dropout).astype(jnp.int32)
    packed = (labels.astype(jnp.int32) | (flags << 30)).astype(jnp.int32)
    tableT = embedding_table.T
    tail = embedding_table[_TAIL_BASE:].reshape(-1)

    mesh = plsc.VectorSubcoreMesh(core_axis_name="c", subcore_axis_name="s")
    out = pl.kernel(
        _body,
        out_type=jax.ShapeDtypeStruct((_B * _D,), jnp.float32),
        mesh=mesh,
        compiler_params=pltpu.CompilerParams(needs_layout_passes=False),
        scratch_types=[
            pltpu.VMEM((_CHUNK,), jnp.int32),
            pltpu.VMEM((_CHUNK,), jnp.int32),
            pltpu.VMEM((_CAP + _LANES,), jnp.int32),
            pltpu.VMEM((_CAP + _LANES,), jnp.int32),
            pltpu.VMEM((_TAIL_N * _D,), jnp.float32),
            pltpu.VMEM((_D, _WIN), jnp.float32),
            pltpu.VMEM((_D, _WIN), jnp.float32),
            pltpu.VMEM((_D, _WIN), jnp.float32),
            pltpu.VMEM((_D, _WIN), jnp.float32),
            pltpu.VMEM((_CAP * _D,), jnp.float32),
            pltpu.VMEM((_PCAP + _LANES,), jnp.int32),
            pltpu.VMEM((_PCAP + _LANES,), jnp.int32),
            pltpu.SemaphoreType.DMA,
            pltpu.SemaphoreType.DMA,
            pltpu.SemaphoreType.DMA,
            pltpu.SemaphoreType.DMA,
            pltpu.SemaphoreType.DMA,
            pltpu.SemaphoreType.DMA,
        ],
    )(packed, tableT, tail)
    return out.reshape(b, _D)
